# baseline (speedup 1.0000x reference)
import functools

import jax
import jax.numpy as jnp
from jax import lax
from jax.experimental import pallas as pl
from jax.experimental.pallas import tpu as pltpu
from jax.experimental.pallas import tpu_sc as plsc

NC = 2
NS = 16
NW = NC * NS
B = 16384
D = 64
ROWS_PER_W = B // NW
CHUNK = 128
NCHUNK = ROWS_PER_W // CHUNK
INV_B = 1.0 / B


def _body(labels_hbm, features_hbm, proxy_hbm, out_hbm,
          idx_v, feat_v, rows_v, acc_v, gsem, fsem):
    wid = lax.axis_index("s") * NC + lax.axis_index("c")
    base = wid * ROWS_PER_W

    pltpu.sync_copy(labels_hbm.at[pl.ds(wid * NCHUNK, NCHUNK)], idx_v)

    accs = [jnp.zeros((16,), jnp.float32) for _ in range(4)]
    for j in range(NCHUNK):
        g = pltpu.async_copy(proxy_hbm.at[idx_v.at[j]], rows_v, gsem)
        f = pltpu.async_copy(
            features_hbm.at[pl.ds(base + j * CHUNK, CHUNK)], feat_v, fsem)
        g.wait()
        f.wait()

        def row_body(r, carry):
            outs = []
            for c in range(4):
                fv = feat_v[r, pl.ds(c * 16, 16)]
                pv = rows_v[r, pl.ds(c * 16, 16)]
                d = jnp.abs(fv - pv)
                m = jnp.minimum(d, 1.0)
                outs.append(carry[c] + m * (d - 0.5 * m))
            return tuple(outs)

        accs = list(lax.fori_loop(0, CHUNK, row_body, tuple(accs)))

    acc_v[...] = (accs[0] + accs[1] + accs[2] + accs[3]) * INV_B
    pltpu.sync_copy(acc_v, out_hbm.at[wid])


@jax.jit
def kernel(features, proxy, labels):
    labels2d = labels.astype(jnp.int32).reshape(B // CHUNK, CHUNK)
    run = pl.kernel(
        _body,
        out_type=jax.ShapeDtypeStruct((NW, 16), jnp.float32),
        mesh=plsc.VectorSubcoreMesh(core_axis_name="c", subcore_axis_name="s"),
        compiler_params=pltpu.CompilerParams(use_tc_tiling_on_sc=False),
        scratch_types=[
            pltpu.VMEM((NCHUNK, CHUNK), jnp.int32),
            pltpu.VMEM((CHUNK, D), jnp.float32),
            pltpu.VMEM((CHUNK, D), jnp.float32),
            pltpu.VMEM((16,), jnp.float32),
            pltpu.SemaphoreType.DMA,
            pltpu.SemaphoreType.DMA,
        ],
    )
    partials = run(labels2d, features, proxy)
    return jnp.sum(partials)
